# baseline (speedup 1.0000x reference)
import functools

import jax
import jax.numpy as jnp
from jax import lax
from jax.experimental import pallas as pl
from jax.experimental.pallas import tpu as pltpu
from jax.experimental.pallas import tpu_sc as plsc

EMBED_DIM = 128
NUM_ROWS = 100
BATCH = 16384

_info = plsc.get_sparse_core_info()
_NC, _NS = _info.num_cores, _info.num_subcores
_NW = _NC * _NS
_BPW = BATCH // _NW
_CHUNK = 128
_NCHUNK = _BPW // _CHUNK


def _project_body(t_ref, w_ref, b_ref, o_ref):
    o_ref[...] = lax.dot_general(
        t_ref[...], w_ref[...],
        dimension_numbers=(((1,), (1,)), ((), ())),
        preferred_element_type=jnp.float32,
        precision=lax.Precision.HIGHEST,
    ) + b_ref[...]


def _project(table, W, b2d):
    return pl.pallas_call(
        _project_body,
        out_shape=jax.ShapeDtypeStruct((table.shape[0], EMBED_DIM), jnp.float32),
    )(table, W, b2d)


_mesh = plsc.VectorSubcoreMesh(core_axis_name="c", subcore_axis_name="s")


@functools.partial(
    pl.kernel,
    out_type=jax.ShapeDtypeStruct((BATCH, EMBED_DIM), jnp.float32),
    mesh=_mesh,
    scratch_types=[
        pltpu.VMEM((_BPW,), jnp.int32),
        pltpu.VMEM((_BPW, EMBED_DIM), jnp.float32),
        pltpu.VMEM_SHARED((NUM_ROWS, EMBED_DIM), jnp.float32),
        pltpu.SemaphoreType.DMA,
    ],
)
def _gather(idx_hbm, ptab_hbm, out_hbm, idx_v, rows_v, stab, sem):
    wid = lax.axis_index("s") * _NC + lax.axis_index("c")
    base = wid * _BPW
    @pl.when(lax.axis_index("s") == 0)
    def _():
        pltpu.sync_copy(ptab_hbm, stab)
    pltpu.sync_copy(idx_hbm.at[pl.ds(base, _BPW)], idx_v)
    plsc.subcore_barrier()
    pltpu.async_copy(stab.at[idx_v], rows_v, sem).wait()
    pltpu.sync_copy(rows_v, out_hbm.at[pl.ds(base, _BPW)])


def kernel(type_info, type_embeddings, W, b):
    idx = type_info.astype(jnp.int32)
    ptab = _project(type_embeddings, W, b.reshape(1, EMBED_DIM))
    out = _gather(idx, ptab)
    return out * (1.0 + b[0])


# speedup vs baseline: 1.2650x; 1.2650x over previous
import functools

import jax
import jax.numpy as jnp
from jax import lax
from jax.experimental import pallas as pl
from jax.experimental.pallas import tpu as pltpu
from jax.experimental.pallas import tpu_sc as plsc

EMBED_DIM = 128
NUM_ROWS = 100
BATCH = 16384

_info = plsc.get_sparse_core_info()
_NC, _NS = _info.num_cores, _info.num_subcores
_NW = _NC * _NS
_BPW = BATCH // _NW
_CHUNK = 64
_NCHUNK = _BPW // _CHUNK


def _project_body(t_ref, w_ref, b_ref, o_ref):
    o_ref[...] = lax.dot_general(
        t_ref[...], w_ref[...],
        dimension_numbers=(((1,), (1,)), ((), ())),
        preferred_element_type=jnp.float32,
        precision=lax.Precision.HIGHEST,
    ) + b_ref[...]


def _project(table, W, b2d):
    return pl.pallas_call(
        _project_body,
        out_shape=jax.ShapeDtypeStruct((table.shape[0], EMBED_DIM), jnp.float32),
    )(table, W, b2d)


_mesh = plsc.VectorSubcoreMesh(core_axis_name="c", subcore_axis_name="s")


@functools.partial(
    pl.kernel,
    out_type=jax.ShapeDtypeStruct((BATCH, EMBED_DIM), jnp.float32),
    mesh=_mesh,
    scratch_types=[
        pltpu.VMEM((_NCHUNK, _CHUNK), jnp.int32),
        pltpu.VMEM((_BPW, EMBED_DIM), jnp.float32),
        pltpu.VMEM_SHARED((NUM_ROWS, EMBED_DIM), jnp.float32),
        [pltpu.SemaphoreType.DMA] * _NCHUNK,
        pltpu.SemaphoreType.DMA,
        pltpu.SemaphoreType.DMA,
    ],
)
def _gather(idx_hbm, ptab_hbm, out_hbm, idx_v, rows_v, stab, gsems, osem, tsem):
    wid = lax.axis_index("s") * _NC + lax.axis_index("c")
    base = wid * _BPW
    stage = lax.axis_index("s") == 0
    @pl.when(stage)
    def _():
        pltpu.make_async_copy(ptab_hbm, stab, tsem).start()
    pltpu.sync_copy(idx_hbm.at[pl.ds(wid * _NCHUNK, _NCHUNK)], idx_v)
    @pl.when(stage)
    def _():
        pltpu.make_async_copy(ptab_hbm, stab, tsem).wait()
    plsc.subcore_barrier()
    gathers = [
        pltpu.make_async_copy(
            stab.at[idx_v.at[j]],
            rows_v.at[pl.ds(j * _CHUNK, _CHUNK)],
            gsems[j],
        )
        for j in range(_NCHUNK)
    ]
    stores = [
        pltpu.make_async_copy(
            rows_v.at[pl.ds(j * _CHUNK, _CHUNK)],
            out_hbm.at[pl.ds(base + j * _CHUNK, _CHUNK)],
            osem,
        )
        for j in range(_NCHUNK)
    ]
    for g in gathers:
        g.start()
    for j in range(_NCHUNK):
        gathers[j].wait()
        stores[j].start()
    for s in stores:
        s.wait()


def kernel(type_info, type_embeddings, W, b):
    idx = type_info.astype(jnp.int32).reshape(_NW * _NCHUNK, _CHUNK)
    ptab = _project(type_embeddings, W, b.reshape(1, EMBED_DIM))
    return _gather(idx, ptab)


# speedup vs baseline: 1.3038x; 1.0306x over previous
import functools

import jax
import jax.numpy as jnp
from jax import lax
from jax.experimental import pallas as pl
from jax.experimental.pallas import tpu as pltpu
from jax.experimental.pallas import tpu_sc as plsc

EMBED_DIM = 128
NUM_ROWS = 100
BATCH = 16384

_info = plsc.get_sparse_core_info()
_NC, _NS = _info.num_cores, _info.num_subcores
_NW = _NC * _NS
_BPW = BATCH // _NW
_CHUNK = 64
_NCHUNK = _BPW // _CHUNK


def _project_body(t_ref, w_ref, b_ref, o_ref):
    o_ref[...] = lax.dot_general(
        t_ref[...], w_ref[...],
        dimension_numbers=(((1,), (1,)), ((), ())),
        preferred_element_type=jnp.float32,
        precision=lax.Precision.HIGHEST,
    ) + b_ref[...]


def _project(table, W, b2d):
    return pl.pallas_call(
        _project_body,
        out_shape=jax.ShapeDtypeStruct((table.shape[0], EMBED_DIM), jnp.float32),
    )(table, W, b2d)


_mesh = plsc.VectorSubcoreMesh(core_axis_name="c", subcore_axis_name="s")


@functools.partial(
    pl.kernel,
    out_type=jax.ShapeDtypeStruct((BATCH, EMBED_DIM), jnp.float32),
    mesh=_mesh,
    scratch_types=[
        pltpu.VMEM((_BPW,), jnp.int32),
        pltpu.VMEM((_BPW, EMBED_DIM), jnp.float32),
        pltpu.VMEM_SHARED((NUM_ROWS, EMBED_DIM), jnp.float32),
        [pltpu.SemaphoreType.DMA] * _NCHUNK,
        pltpu.SemaphoreType.DMA,
        pltpu.SemaphoreType.DMA,
    ],
)
def _gather(idx_hbm, ptab_hbm, out_hbm, idx_v, rows_v, stab, gsems, osem, tsem):
    wid = lax.axis_index("s") * _NC + lax.axis_index("c")
    base = wid * _BPW
    stage = lax.axis_index("s") == 0
    @pl.when(stage)
    def _():
        pltpu.make_async_copy(ptab_hbm, stab, tsem).start()
    pltpu.sync_copy(idx_hbm.at[pl.ds(base, _BPW)], idx_v)
    @pl.when(stage)
    def _():
        pltpu.make_async_copy(ptab_hbm, stab, tsem).wait()
    plsc.subcore_barrier()
    gathers = [
        pltpu.make_async_copy(
            stab.at[idx_v.at[pl.ds(j * _CHUNK, _CHUNK)]],
            rows_v.at[pl.ds(j * _CHUNK, _CHUNK)],
            gsems[j],
        )
        for j in range(_NCHUNK)
    ]
    stores = [
        pltpu.make_async_copy(
            rows_v.at[pl.ds(j * _CHUNK, _CHUNK)],
            out_hbm.at[pl.ds(base + j * _CHUNK, _CHUNK)],
            osem,
        )
        for j in range(_NCHUNK)
    ]
    for g in gathers:
        g.start()
    for j in range(_NCHUNK):
        gathers[j].wait()
        stores[j].start()
    for s in stores:
        s.wait()


def kernel(type_info, type_embeddings, W, b):
    idx = type_info.astype(jnp.int32)
    ptab = _project(type_embeddings, W, b.reshape(1, EMBED_DIM))
    return _gather(idx, ptab)
